# Initial kernel scaffold; baseline (speedup 1.0000x reference)
import jax
import jax.numpy as jnp
from jax.experimental import pallas as pl


def kernel(mol_x, mol_edge_index, mol_batch, pro_x, pro_edge_index, pro_batch, params):
    raise NotImplementedError("write your pallas kernel here")


# speedup vs baseline: 5.4399x; 5.4399x over previous
import functools

import jax
import jax.numpy as jnp
from jax import lax
from jax.experimental import pallas as pl
from jax.experimental.pallas import tpu as pltpu
from jax.experimental.pallas import tpu_sc as plsc

N = 50000
E = 800000
G = 512
D = 32

NC = 2
NS = 16
NW = NC * NS
K = 128
EW = -(-E // (NW * K)) * K
EPAD = EW * NW
NCHUNK = EW // K
NPAD = 51200
RT = NPAD // NS
ZR = 640
NOUT_T = 3128

BR = 2000


def _segsum_body(y_hbm, src_hbm, dst_hbm, out_hbm, sidx, didx, rows, zbuf,
                 acc, sem):
    cid = lax.axis_index("c")
    sid = lax.axis_index("s")
    wid = cid * NS + sid

    def zrow(r, c):
        zbuf[r, pl.ds(0, 16)] = jnp.zeros((16,), jnp.float32)
        zbuf[r, pl.ds(16, 16)] = jnp.zeros((16,), jnp.float32)
        return c

    lax.fori_loop(0, ZR, zrow, 0)

    def zcopy(j, c):
        pltpu.sync_copy(zbuf, acc.at[pl.ds(sid * RT + j * ZR, ZR)])
        return c

    lax.fori_loop(0, RT // ZR, zcopy, 0)
    plsc.subcore_barrier()

    base0 = wid * EW

    def chunk(j, c):
        b = base0 + j * K
        pltpu.sync_copy(src_hbm.at[pl.ds(b, K)], sidx)
        pltpu.sync_copy(dst_hbm.at[pl.ds(b, K)], didx)
        pltpu.async_copy(y_hbm.at[sidx], rows, sem).wait()
        pltpu.sync_copy(rows, acc.at[didx], add=True)
        return c

    lax.fori_loop(0, NCHUNK, chunk, 0)
    plsc.subcore_barrier()

    off = jnp.minimum(sid * NOUT_T, N - NOUT_T)
    pltpu.sync_copy(acc.at[pl.ds(off, NOUT_T)],
                    out_hbm.at[cid, pl.ds(off, NOUT_T)])


_segsum = pl.kernel(
    _segsum_body,
    out_type=jax.ShapeDtypeStruct((NC, N, D), jnp.float32),
    mesh=plsc.VectorSubcoreMesh(core_axis_name="c", subcore_axis_name="s"),
    compiler_params=pltpu.CompilerParams(use_tc_tiling_on_sc=False),
    scratch_types=[
        pltpu.VMEM((K,), jnp.int32),
        pltpu.VMEM((K,), jnp.int32),
        pltpu.VMEM((K, D), jnp.float32),
        pltpu.VMEM((ZR, D), jnp.float32),
        pltpu.VMEM_SHARED((NPAD, D), jnp.float32),
        pltpu.SemaphoreType.DMA,
    ],
)


def _proj_body(x_ref, w_ref, o_ref):
    o_ref[...] = jnp.dot(x_ref[...], w_ref[...],
                         preferred_element_type=jnp.float32)


def _proj(x, w):
    n, din = x.shape
    dout = w.shape[1]
    return pl.pallas_call(
        _proj_body,
        grid=(n // BR,),
        in_specs=[pl.BlockSpec((BR, din), lambda i: (i, 0)),
                  pl.BlockSpec((din, dout), lambda i: (0, 0))],
        out_specs=pl.BlockSpec((BR, dout), lambda i: (i, 0)),
        out_shape=jax.ShapeDtypeStruct((n, dout), jnp.float32),
    )(x, w)


def _layer_core(y_ref, s0_ref, s1_ref, b1_ref, w2_ref, b2_ref, g_ref, bb_ref):
    t = y_ref[...] + s0_ref[...] + s1_ref[...] + b1_ref[...]
    a = jnp.maximum(t, 0.0)
    h = jnp.dot(a, w2_ref[...], preferred_element_type=jnp.float32) + b2_ref[...]
    return jnp.maximum(h, 0.0) * g_ref[...] + bb_ref[...]


def _tail_body(y_ref, s0_ref, s1_ref, b1_ref, w2_ref, b2_ref, g_ref, bb_ref,
               w1n_ref, o_ref):
    x2 = _layer_core(y_ref, s0_ref, s1_ref, b1_ref, w2_ref, b2_ref, g_ref,
                     bb_ref)
    o_ref[...] = jnp.dot(x2, w1n_ref[...], preferred_element_type=jnp.float32)


def _tail(y, s0, s1, b1, w2, b2, g, bb, w1n):
    row = lambda i: (i, 0)
    vec = lambda i: (0, 0)
    return pl.pallas_call(
        _tail_body,
        grid=(N // BR,),
        in_specs=[pl.BlockSpec((BR, D), row), pl.BlockSpec((BR, D), row),
                  pl.BlockSpec((BR, D), row), pl.BlockSpec((1, D), vec),
                  pl.BlockSpec((D, D), vec), pl.BlockSpec((1, D), vec),
                  pl.BlockSpec((1, D), vec), pl.BlockSpec((1, D), vec),
                  pl.BlockSpec((D, D), vec)],
        out_specs=pl.BlockSpec((BR, D), row),
        out_shape=jax.ShapeDtypeStruct((N, D), jnp.float32),
    )(y, s0, s1, b1, w2, b2, g, bb, w1n)


def _tailpool_body(y_ref, s0_ref, s1_ref, b1_ref, w2_ref, b2_ref, g_ref,
                   bb_ref, batch_ref, o_ref):
    x2 = _layer_core(y_ref, s0_ref, s1_ref, b1_ref, w2_ref, b2_ref, g_ref,
                     bb_ref)
    bvec = batch_ref[0, 0, :]
    onehot = (bvec[:, None] == lax.broadcasted_iota(jnp.int32, (BR, G), 1)
              ).astype(jnp.float32)
    contrib = lax.dot_general(onehot, x2, (((0,), (0,)), ((), ())),
                              preferred_element_type=jnp.float32)

    @pl.when(pl.program_id(0) == 0)
    def _():
        o_ref[...] = jnp.zeros_like(o_ref)

    o_ref[...] += contrib


def _tailpool(y, s0, s1, b1, w2, b2, g, bb, batch3):
    row = lambda i: (i, 0)
    vec = lambda i: (0, 0)
    return pl.pallas_call(
        _tailpool_body,
        grid=(N // BR,),
        in_specs=[pl.BlockSpec((BR, D), row), pl.BlockSpec((BR, D), row),
                  pl.BlockSpec((BR, D), row), pl.BlockSpec((1, D), vec),
                  pl.BlockSpec((D, D), vec), pl.BlockSpec((1, D), vec),
                  pl.BlockSpec((1, D), vec), pl.BlockSpec((1, D), vec),
                  pl.BlockSpec((1, 1, BR), lambda i: (i, 0, 0))],
        out_specs=pl.BlockSpec((G, D), vec),
        out_shape=jax.ShapeDtypeStruct((G, D), jnp.float32),
    )(y, s0, s1, b1, w2, b2, g, bb, batch3)


def _head_body(mp_ref, pp_ref, mw1_ref, mb1_ref, mw2_ref, mb2_ref, pw1_ref,
               pb1_ref, pw2_ref, pb2_ref, f1a_ref, f1b_ref, f1b_b_ref,
               f2w_ref, f2b_ref, ow_ref, ob_ref, o_ref):
    dot = functools.partial(jnp.dot, preferred_element_type=jnp.float32)
    m = jnp.maximum(dot(mp_ref[...], mw1_ref[...]) + mb1_ref[...], 0.0)
    m = dot(m, mw2_ref[...]) + mb2_ref[...]
    q = jnp.maximum(dot(pp_ref[...], pw1_ref[...]) + pb1_ref[...], 0.0)
    q = dot(q, pw2_ref[...]) + pb2_ref[...]
    h = jnp.maximum(dot(m, f1a_ref[...]) + dot(q, f1b_ref[...])
                    + f1b_b_ref[...], 0.0)
    h = jnp.maximum(dot(h, f2w_ref[...]) + f2b_ref[...], 0.0)
    o_ref[...] = dot(h, ow_ref[...]) + ob_ref[...]


def _head(mp, pp, p):
    args = (mp, pp,
            p['mol_fcg1_W'], p['mol_fcg1_b'].reshape(1, -1),
            p['mol_fcg2_W'], p['mol_fcg2_b'].reshape(1, -1),
            p['pro_fcg1_W'], p['pro_fcg1_b'].reshape(1, -1),
            p['pro_fcg2_W'], p['pro_fcg2_b'].reshape(1, -1),
            p['fc1_W'][:128], p['fc1_W'][128:], p['fc1_b'].reshape(1, -1),
            p['fc2_W'], p['fc2_b'].reshape(1, -1),
            p['out_W'], p['out_b'].reshape(1, -1))
    return pl.pallas_call(
        _head_body,
        out_shape=jax.ShapeDtypeStruct((G, 1), jnp.float32),
    )(*args)


def _branch(x, edge_index, batch, params, pfx):
    pad = EPAD - E
    src_p = jnp.concatenate(
        [edge_index[0].astype(jnp.int32), jnp.zeros((pad,), jnp.int32)])
    dst_p = jnp.concatenate(
        [edge_index[1].astype(jnp.int32), jnp.full((pad,), N, jnp.int32)])
    batch3 = batch.astype(jnp.int32).reshape(N // BR, 1, BR)

    y = _proj(x, params[f'{pfx}_c1_l1_W'])
    g = params[f'{pfx}_bn1_g'].reshape(1, D)
    bb = params[f'{pfx}_bn1_b'].reshape(1, D)
    for i in range(1, 6):
        s = _segsum(y, src_p, dst_p)
        b1 = params[f'{pfx}_c{i}_l1_b'].reshape(1, D)
        w2 = params[f'{pfx}_c{i}_l2_W']
        b2 = params[f'{pfx}_c{i}_l2_b'].reshape(1, D)
        if i < 5:
            y = _tail(y, s[0], s[1], b1, w2, b2, g, bb,
                      params[f'{pfx}_c{i + 1}_l1_W'])
        else:
            return _tailpool(y, s[0], s[1], b1, w2, b2, g, bb, batch3)


def kernel(mol_x, mol_edge_index, mol_batch, pro_x, pro_edge_index, pro_batch,
           params):
    mp = _branch(mol_x, mol_edge_index, mol_batch, params, 'mol')
    pp = _branch(pro_x, pro_edge_index, pro_batch, params, 'pro')
    return _head(mp, pp, params)


# speedup vs baseline: 6.1993x; 1.1396x over previous
import functools

import jax
import jax.numpy as jnp
from jax import lax
from jax.experimental import pallas as pl
from jax.experimental.pallas import tpu as pltpu
from jax.experimental.pallas import tpu_sc as plsc

N = 50000
E = 800000
G = 512
D = 32

NC = 2
NS = 16
NW = NC * NS
K = 128
NB = 4
NI = 8
CH = 200
NITER = CH // NI
EW = CH * K
EPAD = EW * NW
NPAD = 51200
RT = NPAD // NS
NZC = RT // K
NOUT_T = 3128

BR = 2000


def _segsum_body(y_hbm, src_hbm, dst_hbm, out_hbm, sidx, didx, rows, acc,
                 gsem, ssem, isem, zsem):
    cid = lax.axis_index("c")
    sid = lax.axis_index("s")
    wid = cid * NS + sid
    base = wid * CH

    def zrow(r, c):
        rows[0, r, pl.ds(0, 16)] = jnp.zeros((16,), jnp.float32)
        rows[0, r, pl.ds(16, 16)] = jnp.zeros((16,), jnp.float32)
        return c

    lax.fori_loop(0, K, zrow, 0)
    for j in range(NZC):
        pltpu.async_copy(rows.at[0], acc.at[pl.ds(sid * RT + j * K, K)], zsem)
    for j in range(NZC):
        pltpu.make_async_copy(y_hbm.at[pl.ds(0, K)], rows.at[0], zsem).wait()
    plsc.subcore_barrier()

    def fire_idx(c, slot):
        pltpu.async_copy(src_hbm.at[base + c], sidx.at[slot], isem.at[slot])
        pltpu.async_copy(dst_hbm.at[base + c], didx.at[slot], isem.at[slot])

    def drain(sem_slot, n=1):
        for _ in range(n):
            pltpu.make_async_copy(y_hbm.at[pl.ds(0, K)], rows.at[0],
                                  sem_slot).wait()

    def drain_idx(slot):
        for _ in range(2):
            pltpu.make_async_copy(src_hbm.at[0], sidx.at[slot],
                                  isem.at[slot]).wait()

    fire_idx(0, 0)
    fire_idx(1, 1)

    def oct_(t, carry):
        for k in range(NI):
            c = t * NI + k
            bank = k % NB
            pbank = (k - 1) % NB
            pslot = (k - 1) % NI
            drain_idx(k)
            if k >= NB:
                drain(ssem.at[bank])
            else:
                @pl.when(t > 0)
                def _():
                    drain(ssem.at[bank])
            pltpu.async_copy(y_hbm.at[sidx.at[k]], rows.at[bank],
                             gsem.at[bank])
            if k >= 1:
                drain(gsem.at[pbank])
                pltpu.async_copy(rows.at[pbank], acc.at[didx.at[pslot]],
                                 ssem.at[pbank], add=True)
            else:
                @pl.when(t > 0)
                def _():
                    drain(gsem.at[pbank])
                    pltpu.async_copy(rows.at[pbank], acc.at[didx.at[pslot]],
                                     ssem.at[pbank], add=True)
            if k < NI - 2:
                fire_idx(c + 2, (k + 2) % NI)
            else:
                @pl.when(t < NITER - 1)
                def _():
                    fire_idx(c + 2, (k + 2) % NI)
        return carry

    lax.fori_loop(0, NITER, oct_, 0)
    drain(gsem.at[(NI - 1) % NB])
    pltpu.async_copy(rows.at[(NI - 1) % NB], acc.at[didx.at[NI - 1]],
                     ssem.at[(NI - 1) % NB], add=True)
    for k in range(NB):
        drain(ssem.at[k])
    plsc.subcore_barrier()

    off = jnp.minimum(sid * NOUT_T, N - NOUT_T)
    pltpu.sync_copy(acc.at[pl.ds(off, NOUT_T)],
                    out_hbm.at[cid, pl.ds(off, NOUT_T)])


_segsum = pl.kernel(
    _segsum_body,
    out_type=jax.ShapeDtypeStruct((NC, N, D), jnp.float32),
    mesh=plsc.VectorSubcoreMesh(core_axis_name="c", subcore_axis_name="s"),
    compiler_params=pltpu.CompilerParams(use_tc_tiling_on_sc=False),
    scratch_types=[
        pltpu.VMEM((NI, K), jnp.int32),
        pltpu.VMEM((NI, K), jnp.int32),
        pltpu.VMEM((NB, K, D), jnp.float32),
        pltpu.VMEM_SHARED((NPAD, D), jnp.float32),
        pltpu.SemaphoreType.DMA((NB,)),
        pltpu.SemaphoreType.DMA((NB,)),
        pltpu.SemaphoreType.DMA((NI,)),
        pltpu.SemaphoreType.DMA,
    ],
)


def _proj_body(x_ref, w_ref, o_ref):
    o_ref[...] = jnp.dot(x_ref[...], w_ref[...],
                         preferred_element_type=jnp.float32)


def _proj(x, w):
    n, din = x.shape
    dout = w.shape[1]
    return pl.pallas_call(
        _proj_body,
        grid=(n // BR,),
        in_specs=[pl.BlockSpec((BR, din), lambda i: (i, 0)),
                  pl.BlockSpec((din, dout), lambda i: (0, 0))],
        out_specs=pl.BlockSpec((BR, dout), lambda i: (i, 0)),
        out_shape=jax.ShapeDtypeStruct((n, dout), jnp.float32),
    )(x, w)


def _layer_core(y_ref, s0_ref, s1_ref, b1_ref, w2_ref, b2_ref, g_ref, bb_ref):
    t = y_ref[...] + s0_ref[...] + s1_ref[...] + b1_ref[...]
    a = jnp.maximum(t, 0.0)
    h = jnp.dot(a, w2_ref[...], preferred_element_type=jnp.float32) + b2_ref[...]
    return jnp.maximum(h, 0.0) * g_ref[...] + bb_ref[...]


def _tail_body(y_ref, s0_ref, s1_ref, b1_ref, w2_ref, b2_ref, g_ref, bb_ref,
               w1n_ref, o_ref):
    x2 = _layer_core(y_ref, s0_ref, s1_ref, b1_ref, w2_ref, b2_ref, g_ref,
                     bb_ref)
    o_ref[...] = jnp.dot(x2, w1n_ref[...], preferred_element_type=jnp.float32)


def _tail(y, s0, s1, b1, w2, b2, g, bb, w1n):
    row = lambda i: (i, 0)
    vec = lambda i: (0, 0)
    return pl.pallas_call(
        _tail_body,
        grid=(N // BR,),
        in_specs=[pl.BlockSpec((BR, D), row), pl.BlockSpec((BR, D), row),
                  pl.BlockSpec((BR, D), row), pl.BlockSpec((1, D), vec),
                  pl.BlockSpec((D, D), vec), pl.BlockSpec((1, D), vec),
                  pl.BlockSpec((1, D), vec), pl.BlockSpec((1, D), vec),
                  pl.BlockSpec((D, D), vec)],
        out_specs=pl.BlockSpec((BR, D), row),
        out_shape=jax.ShapeDtypeStruct((N, D), jnp.float32),
    )(y, s0, s1, b1, w2, b2, g, bb, w1n)


def _tailpool_body(y_ref, s0_ref, s1_ref, b1_ref, w2_ref, b2_ref, g_ref,
                   bb_ref, batch_ref, o_ref):
    x2 = _layer_core(y_ref, s0_ref, s1_ref, b1_ref, w2_ref, b2_ref, g_ref,
                     bb_ref)
    bvec = batch_ref[0, 0, :]
    onehot = (bvec[:, None] == lax.broadcasted_iota(jnp.int32, (BR, G), 1)
              ).astype(jnp.float32)
    contrib = lax.dot_general(onehot, x2, (((0,), (0,)), ((), ())),
                              preferred_element_type=jnp.float32)

    @pl.when(pl.program_id(0) == 0)
    def _():
        o_ref[...] = jnp.zeros_like(o_ref)

    o_ref[...] += contrib


def _tailpool(y, s0, s1, b1, w2, b2, g, bb, batch3):
    row = lambda i: (i, 0)
    vec = lambda i: (0, 0)
    return pl.pallas_call(
        _tailpool_body,
        grid=(N // BR,),
        in_specs=[pl.BlockSpec((BR, D), row), pl.BlockSpec((BR, D), row),
                  pl.BlockSpec((BR, D), row), pl.BlockSpec((1, D), vec),
                  pl.BlockSpec((D, D), vec), pl.BlockSpec((1, D), vec),
                  pl.BlockSpec((1, D), vec), pl.BlockSpec((1, D), vec),
                  pl.BlockSpec((1, 1, BR), lambda i: (i, 0, 0))],
        out_specs=pl.BlockSpec((G, D), vec),
        out_shape=jax.ShapeDtypeStruct((G, D), jnp.float32),
    )(y, s0, s1, b1, w2, b2, g, bb, batch3)


def _head_body(mp_ref, pp_ref, mw1_ref, mb1_ref, mw2_ref, mb2_ref, pw1_ref,
               pb1_ref, pw2_ref, pb2_ref, f1a_ref, f1b_ref, f1b_b_ref,
               f2w_ref, f2b_ref, ow_ref, ob_ref, o_ref):
    dot = functools.partial(jnp.dot, preferred_element_type=jnp.float32)
    m = jnp.maximum(dot(mp_ref[...], mw1_ref[...]) + mb1_ref[...], 0.0)
    m = dot(m, mw2_ref[...]) + mb2_ref[...]
    q = jnp.maximum(dot(pp_ref[...], pw1_ref[...]) + pb1_ref[...], 0.0)
    q = dot(q, pw2_ref[...]) + pb2_ref[...]
    h = jnp.maximum(dot(m, f1a_ref[...]) + dot(q, f1b_ref[...])
                    + f1b_b_ref[...], 0.0)
    h = jnp.maximum(dot(h, f2w_ref[...]) + f2b_ref[...], 0.0)
    o_ref[...] = dot(h, ow_ref[...]) + ob_ref[...]


def _head(mp, pp, p):
    args = (mp, pp,
            p['mol_fcg1_W'], p['mol_fcg1_b'].reshape(1, -1),
            p['mol_fcg2_W'], p['mol_fcg2_b'].reshape(1, -1),
            p['pro_fcg1_W'], p['pro_fcg1_b'].reshape(1, -1),
            p['pro_fcg2_W'], p['pro_fcg2_b'].reshape(1, -1),
            p['fc1_W'][:128], p['fc1_W'][128:], p['fc1_b'].reshape(1, -1),
            p['fc2_W'], p['fc2_b'].reshape(1, -1),
            p['out_W'], p['out_b'].reshape(1, -1))
    return pl.pallas_call(
        _head_body,
        out_shape=jax.ShapeDtypeStruct((G, 1), jnp.float32),
    )(*args)


def _branch(x, edge_index, batch, params, pfx):
    pad = EPAD - E
    src_p = jnp.concatenate(
        [edge_index[0].astype(jnp.int32),
         jnp.zeros((pad,), jnp.int32)]).reshape(EPAD // K, K)
    dst_p = jnp.concatenate(
        [edge_index[1].astype(jnp.int32),
         jnp.full((pad,), N, jnp.int32)]).reshape(EPAD // K, K)
    batch3 = batch.astype(jnp.int32).reshape(N // BR, 1, BR)

    y = _proj(x, params[f'{pfx}_c1_l1_W'])
    g = params[f'{pfx}_bn1_g'].reshape(1, D)
    bb = params[f'{pfx}_bn1_b'].reshape(1, D)
    for i in range(1, 6):
        s = _segsum(y, src_p, dst_p)
        b1 = params[f'{pfx}_c{i}_l1_b'].reshape(1, D)
        w2 = params[f'{pfx}_c{i}_l2_W']
        b2 = params[f'{pfx}_c{i}_l2_b'].reshape(1, D)
        if i < 5:
            y = _tail(y, s[0], s[1], b1, w2, b2, g, bb,
                      params[f'{pfx}_c{i + 1}_l1_W'])
        else:
            return _tailpool(y, s[0], s[1], b1, w2, b2, g, bb, batch3)


def kernel(mol_x, mol_edge_index, mol_batch, pro_x, pro_edge_index, pro_batch,
           params):
    mp = _branch(mol_x, mol_edge_index, mol_batch, params, 'mol')
    pp = _branch(pro_x, pro_edge_index, pro_batch, params, 'pro')
    return _head(mp, pp, params)


# speedup vs baseline: 6.8972x; 1.1126x over previous
import functools

import jax
import jax.numpy as jnp
from jax import lax
from jax.experimental import pallas as pl
from jax.experimental.pallas import tpu as pltpu
from jax.experimental.pallas import tpu_sc as plsc

N = 50000
E = 800000
G = 512
D = 32
DIN = 80

NC = 2
NS = 16
K = 128
NB = 4
NI = 8
CH = 400
NITER = CH // NI
EW = CH * K
EPB = EW * NS // K
EPAD = EW * NS
NPAD = 51200
RT = NPAD // NS
NZC = RT // K
NOUT_T = 3128

BR = 2000
NBLK = N // BR


def _segsum_body(y_hbm, src_hbm, dst_hbm, out_hbm, sidx, didx, rows, acc,
                 gsem, ssem, isem, zsem):
    cid = lax.axis_index("c")
    sid = lax.axis_index("s")
    base = cid * EPB + sid * CH

    def zrow(r, c):
        rows[0, r, pl.ds(0, 16)] = jnp.zeros((16,), jnp.float32)
        rows[0, r, pl.ds(16, 16)] = jnp.zeros((16,), jnp.float32)
        return c

    lax.fori_loop(0, K, zrow, 0)
    for j in range(NZC):
        pltpu.async_copy(rows.at[0], acc.at[pl.ds(sid * RT + j * K, K)], zsem)
    for j in range(NZC):
        pltpu.make_async_copy(y_hbm.at[pl.ds(0, K)], rows.at[0], zsem).wait()
    plsc.subcore_barrier()

    def fire_idx(c, slot):
        pltpu.async_copy(src_hbm.at[base + c], sidx.at[slot], isem.at[slot])
        pltpu.async_copy(dst_hbm.at[base + c], didx.at[slot], isem.at[slot])

    def drain(sem_slot):
        pltpu.make_async_copy(y_hbm.at[pl.ds(0, K)], rows.at[0],
                              sem_slot).wait()

    def drain_idx(slot):
        for _ in range(2):
            pltpu.make_async_copy(src_hbm.at[0], sidx.at[slot],
                                  isem.at[slot]).wait()

    fire_idx(0, 0)
    fire_idx(1, 1)

    def oct_(t, carry):
        for k in range(NI):
            c = t * NI + k
            bank = k % NB
            pbank = (k - 1) % NB
            pslot = (k - 1) % NI
            drain_idx(k)
            if k >= NB:
                drain(ssem.at[bank])
            else:
                @pl.when(t > 0)
                def _():
                    drain(ssem.at[bank])
            pltpu.async_copy(y_hbm.at[sidx.at[k]], rows.at[bank],
                             gsem.at[bank])
            if k >= 1:
                drain(gsem.at[pbank])
                pltpu.async_copy(rows.at[pbank], acc.at[didx.at[pslot]],
                                 ssem.at[pbank], add=True)
            else:
                @pl.when(t > 0)
                def _():
                    drain(gsem.at[pbank])
                    pltpu.async_copy(rows.at[pbank], acc.at[didx.at[pslot]],
                                     ssem.at[pbank], add=True)
            if k < NI - 2:
                fire_idx(c + 2, (k + 2) % NI)
            else:
                @pl.when(t < NITER - 1)
                def _():
                    fire_idx(c + 2, (k + 2) % NI)
        return carry

    lax.fori_loop(0, NITER, oct_, 0)
    drain(gsem.at[(NI - 1) % NB])
    pltpu.async_copy(rows.at[(NI - 1) % NB], acc.at[didx.at[NI - 1]],
                     ssem.at[(NI - 1) % NB], add=True)
    for k in range(NB):
        drain(ssem.at[k])
    plsc.subcore_barrier()

    off = jnp.minimum(sid * NOUT_T, N - NOUT_T)
    pltpu.sync_copy(acc.at[pl.ds(off, NOUT_T)],
                    out_hbm.at[cid, pl.ds(off, NOUT_T)])


_SEGSUM = None


def _segsum(y, src, dst):
    global _SEGSUM
    if _SEGSUM is None:
        _SEGSUM = pl.kernel(
            _segsum_body,
            out_type=jax.ShapeDtypeStruct((NC, N, D), jnp.float32),
            mesh=plsc.VectorSubcoreMesh(core_axis_name="c",
                                        subcore_axis_name="s"),
            compiler_params=pltpu.CompilerParams(use_tc_tiling_on_sc=False),
            scratch_types=[
                pltpu.VMEM((NI, K), jnp.int32),
                pltpu.VMEM((NI, K), jnp.int32),
                pltpu.VMEM((NB, K, D), jnp.float32),
                pltpu.VMEM_SHARED((NPAD, D), jnp.float32),
                pltpu.SemaphoreType.DMA((NB,)),
                pltpu.SemaphoreType.DMA((NB,)),
                pltpu.SemaphoreType.DMA((NI,)),
                pltpu.SemaphoreType.DMA,
            ],
        )
    return _SEGSUM(y, src, dst)


_row = lambda i: (i, 0)
_per_branch = lambda i: (i // NBLK, 0, 0)


def _proj_body(x_ref, w_ref, o_ref):
    o_ref[...] = jnp.dot(x_ref[0], w_ref[0],
                         preferred_element_type=jnp.float32)


def _proj(x_all, w_all):
    return pl.pallas_call(
        _proj_body,
        grid=(NC * NBLK,),
        in_specs=[
            pl.BlockSpec((1, BR, DIN), lambda i: (i // NBLK, i % NBLK, 0)),
            pl.BlockSpec((1, DIN, D), _per_branch),
        ],
        out_specs=pl.BlockSpec((BR, D), _row),
        out_shape=jax.ShapeDtypeStruct((NC * N, D), jnp.float32),
    )(x_all, w_all)


def _tail1_body(y_ref, s_ref, b1_ref, w2_ref, b2_ref, g_ref, bb_ref, o_ref):
    a = jnp.maximum(y_ref[...] + s_ref[...] + b1_ref[0], 0.0)
    h = jnp.dot(a, w2_ref[0], preferred_element_type=jnp.float32) + b2_ref[0]
    o_ref[...] = jnp.maximum(h, 0.0) * g_ref[0] + bb_ref[0]


def _tail1(y, s, b1, w2, b2, g, bb):
    return pl.pallas_call(
        _tail1_body,
        grid=(NC * NBLK,),
        in_specs=[pl.BlockSpec((BR, D), _row), pl.BlockSpec((BR, D), _row),
                  pl.BlockSpec((1, 1, D), _per_branch),
                  pl.BlockSpec((1, D, D), _per_branch),
                  pl.BlockSpec((1, 1, D), _per_branch),
                  pl.BlockSpec((1, 1, D), _per_branch),
                  pl.BlockSpec((1, 1, D), _per_branch)],
        out_specs=pl.BlockSpec((BR, D), _row),
        out_shape=jax.ShapeDtypeStruct((NC * N, D), jnp.float32),
    )(y, s, b1, w2, b2, g, bb)


def _tailx_core(x_ref, s_ref, w1_ref, b1_ref, w2_ref, b2_ref, g_ref, bb_ref):
    t = x_ref[...] + s_ref[...]
    a = jnp.maximum(
        jnp.dot(t, w1_ref[0], preferred_element_type=jnp.float32) + b1_ref[0],
        0.0)
    h = jnp.dot(a, w2_ref[0], preferred_element_type=jnp.float32) + b2_ref[0]
    return jnp.maximum(h, 0.0) * g_ref[0] + bb_ref[0]


def _tailx_body(x_ref, s_ref, w1_ref, b1_ref, w2_ref, b2_ref, g_ref, bb_ref,
                o_ref):
    o_ref[...] = _tailx_core(x_ref, s_ref, w1_ref, b1_ref, w2_ref, b2_ref,
                             g_ref, bb_ref)


_TAIL_SPECS = [pl.BlockSpec((BR, D), _row), pl.BlockSpec((BR, D), _row),
               pl.BlockSpec((1, D, D), _per_branch),
               pl.BlockSpec((1, 1, D), _per_branch),
               pl.BlockSpec((1, D, D), _per_branch),
               pl.BlockSpec((1, 1, D), _per_branch),
               pl.BlockSpec((1, 1, D), _per_branch),
               pl.BlockSpec((1, 1, D), _per_branch)]


def _tailx(x, s, w1, b1, w2, b2, g, bb):
    return pl.pallas_call(
        _tailx_body,
        grid=(NC * NBLK,),
        in_specs=_TAIL_SPECS,
        out_specs=pl.BlockSpec((BR, D), _row),
        out_shape=jax.ShapeDtypeStruct((NC * N, D), jnp.float32),
    )(x, s, w1, b1, w2, b2, g, bb)


def _tailpool_body(x_ref, s_ref, w1_ref, b1_ref, w2_ref, b2_ref, g_ref,
                   bb_ref, batch_ref, o_ref):
    x5 = _tailx_core(x_ref, s_ref, w1_ref, b1_ref, w2_ref, b2_ref, g_ref,
                     bb_ref)
    bvec = batch_ref[0, 0, :]
    onehot = (bvec[:, None] == lax.broadcasted_iota(jnp.int32, (BR, G), 1)
              ).astype(jnp.float32)
    hi = x5.astype(jnp.bfloat16).astype(jnp.float32)
    r1 = x5 - hi
    mid = r1.astype(jnp.bfloat16).astype(jnp.float32)
    lo = r1 - mid
    dims = (((0,), (0,)), ((), ()))
    contrib = (lax.dot_general(onehot, hi, dims,
                               preferred_element_type=jnp.float32)
               + lax.dot_general(onehot, mid, dims,
                                 preferred_element_type=jnp.float32)
               + lax.dot_general(onehot, lo, dims,
                                 preferred_element_type=jnp.float32))

    @pl.when(pl.program_id(0) % NBLK == 0)
    def _():
        o_ref[...] = jnp.zeros_like(o_ref)

    o_ref[...] += contrib


def _tailpool(x, s, w1, b1, w2, b2, g, bb, batch3):
    return pl.pallas_call(
        _tailpool_body,
        grid=(NC * NBLK,),
        in_specs=_TAIL_SPECS + [pl.BlockSpec((1, 1, BR),
                                             lambda i: (i, 0, 0))],
        out_specs=pl.BlockSpec((1, G, D), _per_branch),
        out_shape=jax.ShapeDtypeStruct((NC, G, D), jnp.float32),
    )(x, s, w1, b1, w2, b2, g, bb, batch3)


def _head_body(mp_ref, pp_ref, mw1_ref, mb1_ref, mw2_ref, mb2_ref, pw1_ref,
               pb1_ref, pw2_ref, pb2_ref, f1w_ref, f1b_b_ref,
               f2w_ref, f2b_ref, ow_ref, ob_ref, o_ref):
    dot = functools.partial(jnp.dot, preferred_element_type=jnp.float32)
    m = jnp.maximum(dot(mp_ref[...], mw1_ref[...]) + mb1_ref[...], 0.0)
    m = dot(m, mw2_ref[...]) + mb2_ref[...]
    q = jnp.maximum(dot(pp_ref[...], pw1_ref[...]) + pb1_ref[...], 0.0)
    q = dot(q, pw2_ref[...]) + pb2_ref[...]
    xc = jnp.concatenate([m, q], axis=1)
    h = jnp.maximum(dot(xc, f1w_ref[...]) + f1b_b_ref[...], 0.0)
    h = jnp.maximum(dot(h, f2w_ref[...]) + f2b_ref[...], 0.0)
    o_ref[...] = dot(h, ow_ref[...]) + ob_ref[...]


def _head(mp, pp, p):
    args = (mp, pp,
            p['mol_fcg1_W'], p['mol_fcg1_b'].reshape(1, -1),
            p['mol_fcg2_W'], p['mol_fcg2_b'].reshape(1, -1),
            p['pro_fcg1_W'], p['pro_fcg1_b'].reshape(1, -1),
            p['pro_fcg2_W'], p['pro_fcg2_b'].reshape(1, -1),
            p['fc1_W'], p['fc1_b'].reshape(1, -1),
            p['fc2_W'], p['fc2_b'].reshape(1, -1),
            p['out_W'], p['out_b'].reshape(1, -1))
    return pl.pallas_call(
        _head_body,
        out_shape=jax.ShapeDtypeStruct((G, 1), jnp.float32),
    )(*args)


def _edges(edge_index, offset):
    pad = EPAD - E
    src = jnp.concatenate(
        [edge_index[0].astype(jnp.int32) + offset,
         jnp.full((pad,), N, jnp.int32)]).reshape(EPB, K)
    dst = jnp.concatenate(
        [edge_index[1].astype(jnp.int32),
         jnp.full((pad,), N, jnp.int32)]).reshape(EPB, K)
    return src, dst


def _stackp(params, name, shape):
    return jnp.stack([params[f'mol_{name}'].reshape(shape),
                      params[f'pro_{name}'].reshape(shape)])


def kernel(mol_x, mol_edge_index, mol_batch, pro_x, pro_edge_index, pro_batch,
           params):
    p = params
    msrc, mdst = _edges(mol_edge_index, 0)
    psrc, pdst = _edges(pro_edge_index, N)
    src_all = jnp.concatenate([msrc, psrc])
    dst_all = jnp.concatenate([mdst, pdst])
    batch3 = jnp.concatenate(
        [mol_batch.astype(jnp.int32).reshape(NBLK, 1, BR),
         pro_batch.astype(jnp.int32).reshape(NBLK, 1, BR)])

    x_all = jnp.stack([jnp.pad(mol_x, ((0, 0), (0, DIN - mol_x.shape[1]))),
                       jnp.pad(pro_x, ((0, 0), (0, DIN - pro_x.shape[1])))])
    w1_all = jnp.stack(
        [jnp.pad(p['mol_c1_l1_W'], ((0, DIN - p['mol_c1_l1_W'].shape[0]),
                                    (0, 0))),
         jnp.pad(p['pro_c1_l1_W'], ((0, DIN - p['pro_c1_l1_W'].shape[0]),
                                    (0, 0)))])

    g = _stackp(p, 'bn1_g', (1, D))
    bb = _stackp(p, 'bn1_b', (1, D))

    y = _proj(x_all, w1_all)
    for i in range(1, 6):
        s = _segsum(y, src_all, dst_all).reshape(NC * N, D)
        b1 = _stackp(p, f'c{i}_l1_b', (1, D))
        w2 = _stackp(p, f'c{i}_l2_W', (D, D))
        b2 = _stackp(p, f'c{i}_l2_b', (1, D))
        if i == 1:
            y = _tail1(y, s, b1, w2, b2, g, bb)
        else:
            w1 = _stackp(p, f'c{i}_l1_W', (D, D))
            if i < 5:
                y = _tailx(y, s, w1, b1, w2, b2, g, bb)
            else:
                pool = _tailpool(y, s, w1, b1, w2, b2, g, bb, batch3)
    return _head(pool[0], pool[1], p)
